# Initial kernel scaffold; baseline (speedup 1.0000x reference)
import jax
import jax.numpy as jnp
from jax.experimental import pallas as pl


def kernel(input_ids, table):
    raise NotImplementedError("write your pallas kernel here")


# speedup vs baseline: 1.3459x; 1.3459x over previous
import functools

import jax
import jax.numpy as jnp
from jax import lax
from jax.experimental import pallas as pl
from jax.experimental.pallas import tpu as pltpu
from jax.experimental.pallas import tpu_sc as plsc

B_ROWS = 4096
B_COLS = 200
EMBD = 32
B_TOTAL = B_ROWS * B_COLS
NC, NS, LANES = 2, 16, 16
NW = NC * NS
N_PER_W = B_TOTAL // NW
GCHUNK = 128
K = 8
BLK = K * GCHUNK
NBLK = N_PER_W // BLK
IDX_ROWS_PER_W = N_PER_W // GCHUNK


def _tanh16(x):
    e = jnp.exp(x + x)
    return 1.0 - 2.0 / (e + 1.0)


@functools.partial(
    pl.kernel,
    out_type=jax.ShapeDtypeStruct((B_TOTAL, EMBD), jnp.float32),
    mesh=plsc.VectorSubcoreMesh(core_axis_name="c", subcore_axis_name="s"),
    scratch_types=[
        pltpu.VMEM((K, GCHUNK), jnp.int32),
        pltpu.VMEM((BLK, EMBD), jnp.float32),
        pltpu.SemaphoreType.DMA,
    ],
    compiler_params=pltpu.CompilerParams(use_tc_tiling_on_sc=False),
)
def _embed_tanh(idx_hbm, table_hbm, out_hbm, idx_v, rows_v, sem):
    wid = lax.axis_index("s") * NC + lax.axis_index("c")
    idx_row0 = wid * IDX_ROWS_PER_W
    out_row0 = wid * N_PER_W

    def block(b, carry):
        pltpu.sync_copy(idx_hbm.at[pl.ds(idx_row0 + b * K, K)], idx_v)
        descs = [
            pltpu.async_copy(
                table_hbm.at[idx_v.at[j]],
                rows_v.at[pl.ds(j * GCHUNK, GCHUNK)],
                sem,
            )
            for j in range(K)
        ]
        for d in descs:
            d.wait()

        @plsc.parallel_loop(0, BLK, step=1, unroll=8)
        def row(i):
            rows_v[i, pl.ds(0, LANES)] = _tanh16(rows_v[i, pl.ds(0, LANES)])
            rows_v[i, pl.ds(LANES, LANES)] = _tanh16(
                rows_v[i, pl.ds(LANES, LANES)]
            )

        pltpu.sync_copy(rows_v, out_hbm.at[pl.ds(out_row0 + b * BLK, BLK)])
        return carry

    lax.fori_loop(0, NBLK, block, 0)


def kernel(input_ids, table):
    idx2 = input_ids.reshape(B_TOTAL // GCHUNK, GCHUNK)
    out = _embed_tanh(idx2, table)
    return out.reshape(B_ROWS, B_COLS, EMBD)
